# baseline (speedup 1.0000x reference)
import functools

import jax
import jax.numpy as jnp
from jax import lax
from jax.experimental import pallas as pl
from jax.experimental.pallas import tpu as pltpu
from jax.experimental.pallas import tpu_sc as plsc

NEG_SLOPE = 0.2
NC = 2
NS = 16
NW = NC * NS
CHUNK = 128
N_BLOCK = 512


def _edge_sc_body(npad, n_chunks, d,
                  h_hbm, src_hbm, dst_hbm, asrc_hbm, adst_hbm,
                  outS_hbm, outD_hbm,
                  asrc_l, adst_l, sidx, didx, e_ref, rows, gsem,
                  acc, den_l):
    c = lax.axis_index("c")
    s = lax.axis_index("s")
    wid = c * NS + s
    rpt = npad // NS

    def _zb_row(j, _):
        for k in range(d // 16):
            rows[j, pl.ds(k * 16, 16)] = jnp.zeros((16,), jnp.float32)
        return 0
    lax.fori_loop(0, CHUNK, _zb_row, 0)

    def _zd(j, _):
        den_l[pl.ds(j * 16, 16)] = jnp.zeros((16,), jnp.float32)
        return 0
    lax.fori_loop(0, npad // 16, _zd, 0)

    for t in range(rpt // CHUNK):
        pltpu.sync_copy(rows, acc.at[pl.ds(s * rpt + t * CHUNK, CHUNK)])

    pltpu.sync_copy(asrc_hbm, asrc_l)
    pltpu.sync_copy(adst_hbm, adst_l)

    plsc.subcore_barrier()

    def _chunk(ci, _):
        base = (wid * n_chunks + ci) * CHUNK
        pltpu.sync_copy(src_hbm.at[pl.ds(base, CHUNK)], sidx)
        pltpu.sync_copy(dst_hbm.at[pl.ds(base, CHUNK)], didx)
        cp = pltpu.async_copy(h_hbm.at[sidx], rows, gsem)
        for j in range(CHUNK // 16):
            si = sidx[pl.ds(j * 16, 16)]
            di = didx[pl.ds(j * 16, 16)]
            t = (plsc.load_gather(asrc_l, [si])
                 + plsc.load_gather(adst_l, [di]))
            ev = jnp.exp(jnp.maximum(t, NEG_SLOPE * t))
            e_ref[pl.ds(j * 16, 16)] = ev
            plsc.addupdate_scatter(den_l, [di], ev)
        cp.wait()

        def _scale(r, _):
            ev = plsc.load_gather(
                e_ref, [lax.broadcast_in_dim(r, (16,), ())])
            for k in range(d // 16):
                rows[r, pl.ds(k * 16, 16)] = rows[r, pl.ds(k * 16, 16)] * ev
            return 0
        lax.fori_loop(0, CHUNK, _scale, 0)

        pltpu.sync_copy(rows, acc.at[didx], add=True)
        return 0
    lax.fori_loop(0, n_chunks, _chunk, 0)

    plsc.subcore_barrier()
    pltpu.sync_copy(acc.at[pl.ds(s * rpt, rpt)],
                    outS_hbm.at[pl.ds(c * npad + s * rpt, rpt)])
    pltpu.sync_copy(den_l, outD_hbm.at[pl.ds(wid * npad, npad)])


def _edge_stage(h_pad, asrc_pad, adst_pad, src_pad, dst_pad):
    npad, d = h_pad.shape
    e_tot = src_pad.shape[0]
    epw = e_tot // NW
    n_chunks = epw // CHUNK
    mesh = plsc.VectorSubcoreMesh(core_axis_name="c", subcore_axis_name="s")

    outS, outD = pl.kernel(
        functools.partial(_edge_sc_body, npad, n_chunks, d),
        out_type=[jax.ShapeDtypeStruct((NC * npad, d), jnp.float32),
                  jax.ShapeDtypeStruct((NW * npad,), jnp.float32)],
        mesh=mesh,
        compiler_params=pltpu.CompilerParams(needs_layout_passes=False),
        scratch_types=[
            pltpu.VMEM((npad,), jnp.float32),
            pltpu.VMEM((npad,), jnp.float32),
            pltpu.VMEM((CHUNK,), jnp.int32),
            pltpu.VMEM((CHUNK,), jnp.int32),
            pltpu.VMEM((CHUNK,), jnp.float32),
            pltpu.VMEM((CHUNK, d), jnp.float32),
            pltpu.SemaphoreType.DMA,
            pltpu.VMEM_SHARED((npad, d), jnp.float32),
            pltpu.VMEM((npad,), jnp.float32),
        ],
    )(h_pad, src_pad, dst_pad, asrc_pad, adst_pad)
    return outS.reshape(NC, npad, d), outD.reshape(NW, npad, 1)


def _dense1_body(x_ref, w_ref, avs_ref, avd_ref, h_ref, s_ref, d_ref):
    h = jnp.dot(x_ref[...], w_ref[...], preferred_element_type=jnp.float32)
    h_ref[...] = h
    s_ref[...] = (h @ avs_ref[...])[:, None]
    d_ref[...] = (h @ avd_ref[...])[:, None]


def _mid_body(S_ref, D_ref, b_ref, w_ref, avs_ref, avd_ref,
              h_ref, s_ref, d_ref):
    den = jnp.sum(D_ref[...], axis=0)
    z = (S_ref[0] + S_ref[1]) / (den + 1e-16) + b_ref[...]
    z = jnp.maximum(z, 0.0)
    h = jnp.dot(z, w_ref[...], preferred_element_type=jnp.float32)
    h_ref[...] = h
    s_ref[...] = (h @ avs_ref[...])[:, None]
    d_ref[...] = (h @ avd_ref[...])[:, None]


def _final_body(S_ref, D_ref, b_ref, o_ref):
    den = jnp.sum(D_ref[...], axis=0)
    o = (S_ref[0] + S_ref[1]) / (den + 1e-16) + b_ref[...]
    m = jnp.max(o, axis=1, keepdims=True)
    lse = jnp.log(jnp.sum(jnp.exp(o - m), axis=1, keepdims=True))
    o_ref[...] = o - m - lse


def _dense_outs(npad, d):
    return [
        jax.ShapeDtypeStruct((npad, d), jnp.float32),
        jax.ShapeDtypeStruct((npad, 1), jnp.float32),
        jax.ShapeDtypeStruct((npad, 1), jnp.float32),
    ]


def _dense_out_specs():
    return [
        pl.BlockSpec((N_BLOCK, None), lambda i: (i, 0)),
        pl.BlockSpec((N_BLOCK, 1), lambda i: (i, 0)),
        pl.BlockSpec((N_BLOCK, 1), lambda i: (i, 0)),
    ]


def _dense1(x_pad, W, att_src, att_dst):
    npad, d = x_pad.shape
    h, a_s, a_d = pl.pallas_call(
        _dense1_body,
        grid=(npad // N_BLOCK,),
        in_specs=[
            pl.BlockSpec((N_BLOCK, d), lambda i: (i, 0)),
            pl.BlockSpec((d, d), lambda i: (0, 0)),
            pl.BlockSpec((d,), lambda i: (0,)),
            pl.BlockSpec((d,), lambda i: (0,)),
        ],
        out_specs=[
            pl.BlockSpec((N_BLOCK, d), lambda i: (i, 0)),
            pl.BlockSpec((N_BLOCK, 1), lambda i: (i, 0)),
            pl.BlockSpec((N_BLOCK, 1), lambda i: (i, 0)),
        ],
        out_shape=_dense_outs(npad, d),
    )(x_pad, W, att_src, att_dst)
    return h, a_s.reshape(npad), a_d.reshape(npad)


def _mid(S, D, bias, W, att_src, att_dst):
    _, npad, d = S.shape
    h, a_s, a_d = pl.pallas_call(
        _mid_body,
        grid=(npad // N_BLOCK,),
        in_specs=[
            pl.BlockSpec((NC, N_BLOCK, d), lambda i: (0, i, 0)),
            pl.BlockSpec((NW, N_BLOCK, 1), lambda i: (0, i, 0)),
            pl.BlockSpec((d,), lambda i: (0,)),
            pl.BlockSpec((d, d), lambda i: (0, 0)),
            pl.BlockSpec((d,), lambda i: (0,)),
            pl.BlockSpec((d,), lambda i: (0,)),
        ],
        out_specs=[
            pl.BlockSpec((N_BLOCK, d), lambda i: (i, 0)),
            pl.BlockSpec((N_BLOCK, 1), lambda i: (i, 0)),
            pl.BlockSpec((N_BLOCK, 1), lambda i: (i, 0)),
        ],
        out_shape=_dense_outs(npad, d),
    )(S, D, bias, W, att_src, att_dst)
    return h, a_s.reshape(npad), a_d.reshape(npad)


def _final(S, D, bias):
    _, npad, d = S.shape
    return pl.pallas_call(
        _final_body,
        grid=(npad // N_BLOCK,),
        in_specs=[
            pl.BlockSpec((NC, N_BLOCK, d), lambda i: (0, i, 0)),
            pl.BlockSpec((NW, N_BLOCK, 1), lambda i: (0, i, 0)),
            pl.BlockSpec((d,), lambda i: (0,)),
        ],
        out_specs=pl.BlockSpec((N_BLOCK, d), lambda i: (i, 0)),
        out_shape=jax.ShapeDtypeStruct((npad, d), jnp.float32),
    )(S, D, bias)


def kernel(x, edge_index, W1, att_src1, att_dst1, b1, W2, att_src2, att_dst2, b2):
    n, d = x.shape
    e = edge_index.shape[1]
    npad = -(-(n + NS) // (NS * CHUNK)) * (NS * CHUNK)
    e_tot = e + n
    e_pad = -(-e_tot // (NW * CHUNK)) * (NW * CHUNK)

    loop = jnp.arange(n, dtype=edge_index.dtype)
    src = jnp.concatenate(
        [edge_index[0], loop,
         jnp.zeros((e_pad - e_tot,), edge_index.dtype)])
    dst = jnp.concatenate(
        [edge_index[1], loop,
         jnp.full((e_pad - e_tot,), n, edge_index.dtype)])

    x_pad = jnp.pad(x, ((0, npad - n), (0, 0)))
    h1, a1s, a1d = _dense1(x_pad, W1, att_src1, att_dst1)
    S1, D1 = _edge_stage(h1, a1s, a1d, src, dst)
    h2, a2s, a2d = _mid(S1, D1, b1, W2, att_src2, att_dst2)
    S2, D2 = _edge_stage(h2, a2s, a2d, src, dst)
    out = _final(S2, D2, b2)
    return out[:n]


# speedup vs baseline: 1.3328x; 1.3328x over previous
import functools

import jax
import jax.numpy as jnp
from jax import lax
from jax.experimental import pallas as pl
from jax.experimental.pallas import tpu as pltpu
from jax.experimental.pallas import tpu_sc as plsc

NEG_SLOPE = 0.2
NC = 2
NS = 16
NW = NC * NS
CHUNK = 128
N_BLOCK = 512


def _edge_sc_body(npad, n_chunks, d,
                  h_hbm, src_hbm, dst_hbm, asrc_hbm, adst_hbm,
                  outS_hbm, outD_hbm,
                  asrc_l, adst_l, sidx, didx, e_ref, rows, gsem,
                  acc, den):
    c = lax.axis_index("c")
    s = lax.axis_index("s")
    wid = c * NS + s
    rpt = npad // NS

    def _zb_row(j, _):
        for k in range(d // 16):
            rows[j, pl.ds(k * 16, 16)] = jnp.zeros((16,), jnp.float32)
        return 0
    lax.fori_loop(0, CHUNK, _zb_row, 0)
    for j in range(CHUNK // 16):
        e_ref[pl.ds(j * 16, 16)] = jnp.zeros((16,), jnp.float32)

    for t in range(rpt // CHUNK):
        pltpu.sync_copy(rows, acc.at[pl.ds(s * rpt + t * CHUNK, CHUNK)])
        pltpu.sync_copy(e_ref, den.at[pl.ds(s * rpt + t * CHUNK, CHUNK)])

    pltpu.sync_copy(asrc_hbm, asrc_l)
    pltpu.sync_copy(adst_hbm, adst_l)

    plsc.subcore_barrier()

    def _chunk(ci, _):
        base = (wid * n_chunks + ci) * CHUNK
        pltpu.sync_copy(src_hbm.at[pl.ds(base, CHUNK)], sidx)
        pltpu.sync_copy(dst_hbm.at[pl.ds(base, CHUNK)], didx)
        cp = pltpu.async_copy(h_hbm.at[sidx], rows, gsem)
        for j in range(CHUNK // 16):
            si = sidx[pl.ds(j * 16, 16)]
            di = didx[pl.ds(j * 16, 16)]
            t = (plsc.load_gather(asrc_l, [si])
                 + plsc.load_gather(adst_l, [di]))
            e_ref[pl.ds(j * 16, 16)] = jnp.exp(jnp.maximum(t, NEG_SLOPE * t))
        cp.wait()

        def _scale(r, _):
            ev = plsc.load_gather(
                e_ref, [lax.broadcast_in_dim(r, (16,), ())])
            for k in range(d // 16):
                rows[r, pl.ds(k * 16, 16)] = rows[r, pl.ds(k * 16, 16)] * ev
            return 0
        lax.fori_loop(0, CHUNK, _scale, 0)

        pltpu.sync_copy(e_ref, den.at[didx], add=True)
        pltpu.sync_copy(rows, acc.at[didx], add=True)
        return 0
    lax.fori_loop(0, n_chunks, _chunk, 0)

    plsc.subcore_barrier()
    pltpu.sync_copy(acc.at[pl.ds(s * rpt, rpt)],
                    outS_hbm.at[pl.ds(c * npad + s * rpt, rpt)])
    pltpu.sync_copy(den.at[pl.ds(s * rpt, rpt)],
                    outD_hbm.at[pl.ds(c * npad + s * rpt, rpt)])


def _edge_stage(h_pad, asrc_pad, adst_pad, src_pad, dst_pad):
    npad, d = h_pad.shape
    e_tot = src_pad.shape[0]
    epw = e_tot // NW
    n_chunks = epw // CHUNK
    mesh = plsc.VectorSubcoreMesh(core_axis_name="c", subcore_axis_name="s")

    outS, outD = pl.kernel(
        functools.partial(_edge_sc_body, npad, n_chunks, d),
        out_type=[jax.ShapeDtypeStruct((NC * npad, d), jnp.float32),
                  jax.ShapeDtypeStruct((NC * npad,), jnp.float32)],
        mesh=mesh,
        compiler_params=pltpu.CompilerParams(needs_layout_passes=False),
        scratch_types=[
            pltpu.VMEM((npad,), jnp.float32),
            pltpu.VMEM((npad,), jnp.float32),
            pltpu.VMEM((CHUNK,), jnp.int32),
            pltpu.VMEM((CHUNK,), jnp.int32),
            pltpu.VMEM((CHUNK,), jnp.float32),
            pltpu.VMEM((CHUNK, d), jnp.float32),
            pltpu.SemaphoreType.DMA,
            pltpu.VMEM_SHARED((npad, d), jnp.float32),
            pltpu.VMEM_SHARED((npad,), jnp.float32),
        ],
    )(h_pad, src_pad, dst_pad, asrc_pad, adst_pad)
    return outS.reshape(NC, npad, d), outD.reshape(NC, npad, 1)


def _dense1_body(x_ref, w_ref, avs_ref, avd_ref, h_ref, s_ref, d_ref):
    h = jnp.dot(x_ref[...], w_ref[...], preferred_element_type=jnp.float32)
    h_ref[...] = h
    s_ref[...] = (h @ avs_ref[...])[:, None]
    d_ref[...] = (h @ avd_ref[...])[:, None]


def _mid_body(S_ref, D_ref, b_ref, w_ref, avs_ref, avd_ref,
              h_ref, s_ref, d_ref):
    z = (S_ref[0] + S_ref[1]) / (D_ref[0] + D_ref[1] + 1e-16) + b_ref[...]
    z = jnp.maximum(z, 0.0)
    h = jnp.dot(z, w_ref[...], preferred_element_type=jnp.float32)
    h_ref[...] = h
    s_ref[...] = (h @ avs_ref[...])[:, None]
    d_ref[...] = (h @ avd_ref[...])[:, None]


def _final_body(S_ref, D_ref, b_ref, o_ref):
    o = (S_ref[0] + S_ref[1]) / (D_ref[0] + D_ref[1] + 1e-16) + b_ref[...]
    m = jnp.max(o, axis=1, keepdims=True)
    lse = jnp.log(jnp.sum(jnp.exp(o - m), axis=1, keepdims=True))
    o_ref[...] = o - m - lse


def _dense_outs(npad, d):
    return [
        jax.ShapeDtypeStruct((npad, d), jnp.float32),
        jax.ShapeDtypeStruct((npad, 1), jnp.float32),
        jax.ShapeDtypeStruct((npad, 1), jnp.float32),
    ]


def _dense_out_specs():
    return [
        pl.BlockSpec((N_BLOCK, None), lambda i: (i, 0)),
        pl.BlockSpec((N_BLOCK, 1), lambda i: (i, 0)),
        pl.BlockSpec((N_BLOCK, 1), lambda i: (i, 0)),
    ]


def _dense1(x_pad, W, att_src, att_dst):
    npad, d = x_pad.shape
    h, a_s, a_d = pl.pallas_call(
        _dense1_body,
        grid=(npad // N_BLOCK,),
        in_specs=[
            pl.BlockSpec((N_BLOCK, d), lambda i: (i, 0)),
            pl.BlockSpec((d, d), lambda i: (0, 0)),
            pl.BlockSpec((d,), lambda i: (0,)),
            pl.BlockSpec((d,), lambda i: (0,)),
        ],
        out_specs=[
            pl.BlockSpec((N_BLOCK, d), lambda i: (i, 0)),
            pl.BlockSpec((N_BLOCK, 1), lambda i: (i, 0)),
            pl.BlockSpec((N_BLOCK, 1), lambda i: (i, 0)),
        ],
        out_shape=_dense_outs(npad, d),
    )(x_pad, W, att_src, att_dst)
    return h, a_s.reshape(npad), a_d.reshape(npad)


def _mid(S, D, bias, W, att_src, att_dst):
    _, npad, d = S.shape
    h, a_s, a_d = pl.pallas_call(
        _mid_body,
        grid=(npad // N_BLOCK,),
        in_specs=[
            pl.BlockSpec((NC, N_BLOCK, d), lambda i: (0, i, 0)),
            pl.BlockSpec((NC, N_BLOCK, 1), lambda i: (0, i, 0)),
            pl.BlockSpec((d,), lambda i: (0,)),
            pl.BlockSpec((d, d), lambda i: (0, 0)),
            pl.BlockSpec((d,), lambda i: (0,)),
            pl.BlockSpec((d,), lambda i: (0,)),
        ],
        out_specs=[
            pl.BlockSpec((N_BLOCK, d), lambda i: (i, 0)),
            pl.BlockSpec((N_BLOCK, 1), lambda i: (i, 0)),
            pl.BlockSpec((N_BLOCK, 1), lambda i: (i, 0)),
        ],
        out_shape=_dense_outs(npad, d),
    )(S, D, bias, W, att_src, att_dst)
    return h, a_s.reshape(npad), a_d.reshape(npad)


def _final(S, D, bias):
    _, npad, d = S.shape
    return pl.pallas_call(
        _final_body,
        grid=(npad // N_BLOCK,),
        in_specs=[
            pl.BlockSpec((NC, N_BLOCK, d), lambda i: (0, i, 0)),
            pl.BlockSpec((NC, N_BLOCK, 1), lambda i: (0, i, 0)),
            pl.BlockSpec((d,), lambda i: (0,)),
        ],
        out_specs=pl.BlockSpec((N_BLOCK, d), lambda i: (i, 0)),
        out_shape=jax.ShapeDtypeStruct((npad, d), jnp.float32),
    )(S, D, bias)


def kernel(x, edge_index, W1, att_src1, att_dst1, b1, W2, att_src2, att_dst2, b2):
    n, d = x.shape
    e = edge_index.shape[1]
    npad = -(-(n + NS) // (NS * CHUNK)) * (NS * CHUNK)
    e_tot = e + n
    e_pad = -(-e_tot // (NW * CHUNK)) * (NW * CHUNK)

    loop = jnp.arange(n, dtype=edge_index.dtype)
    src = jnp.concatenate(
        [edge_index[0], loop,
         jnp.zeros((e_pad - e_tot,), edge_index.dtype)])
    dst = jnp.concatenate(
        [edge_index[1], loop,
         jnp.full((e_pad - e_tot,), n, edge_index.dtype)])

    x_pad = jnp.pad(x, ((0, npad - n), (0, 0)))
    h1, a1s, a1d = _dense1(x_pad, W1, att_src1, att_dst1)
    S1, D1 = _edge_stage(h1, a1s, a1d, src, dst)
    h2, a2s, a2d = _mid(S1, D1, b1, W2, att_src2, att_dst2)
    S2, D2 = _edge_stage(h2, a2s, a2d, src, dst)
    out = _final(S2, D2, b2)
    return out[:n]


# speedup vs baseline: 1.3588x; 1.0195x over previous
import functools

import jax
import jax.numpy as jnp
from jax import lax
from jax.experimental import pallas as pl
from jax.experimental.pallas import tpu as pltpu
from jax.experimental.pallas import tpu_sc as plsc

NEG_SLOPE = 0.2
NC = 2
NS = 16
NW = NC * NS
CHUNK = 128
N_BLOCK = 512


def _edge_sc_body(npad, n_chunks, d,
                  h_hbm, src_hbm, dst_hbm, asrc_hbm, adst_hbm,
                  outS_hbm, outD_hbm,
                  asrc_l, adst_l, sidx, didx, e_ref, rows, gsem,
                  acc, den):
    c = lax.axis_index("c")
    s = lax.axis_index("s")
    wid = c * NS + s
    rpt = npad // NS

    def _zb_row(j, _):
        for k in range(d // 16):
            rows[j, pl.ds(k * 16, 16)] = jnp.zeros((16,), jnp.float32)
        return 0
    lax.fori_loop(0, CHUNK, _zb_row, 0)
    for j in range(CHUNK // 16):
        e_ref[pl.ds(j * 16, 16)] = jnp.zeros((16,), jnp.float32)

    for t in range(rpt // CHUNK):
        pltpu.sync_copy(rows, acc.at[pl.ds(s * rpt + t * CHUNK, CHUNK)])
        pltpu.sync_copy(e_ref, den.at[pl.ds(s * rpt + t * CHUNK, CHUNK)])

    pltpu.sync_copy(asrc_hbm, asrc_l)
    pltpu.sync_copy(adst_hbm, adst_l)

    plsc.subcore_barrier()

    def _chunk(ci, _):
        base = (wid * n_chunks + ci) * CHUNK
        pltpu.sync_copy(src_hbm.at[pl.ds(base, CHUNK)], sidx)
        pltpu.sync_copy(dst_hbm.at[pl.ds(base, CHUNK)], didx)
        cp = pltpu.async_copy(h_hbm.at[sidx], rows, gsem)
        for j in range(CHUNK // 16):
            si = sidx[pl.ds(j * 16, 16)]
            di = didx[pl.ds(j * 16, 16)]
            t = (plsc.load_gather(asrc_l, [si])
                 + plsc.load_gather(adst_l, [di]))
            e_ref[pl.ds(j * 16, 16)] = jnp.exp(jnp.maximum(t, NEG_SLOPE * t))
        cp.wait()

        def _scale(r4, _):
            for u in range(4):
                r = r4 * 4 + u
                ev = plsc.load_gather(
                    e_ref, [lax.broadcast_in_dim(r, (16,), ())])
                for k in range(d // 16):
                    rows[r, pl.ds(k * 16, 16)] = (
                        rows[r, pl.ds(k * 16, 16)] * ev)
            return 0
        lax.fori_loop(0, CHUNK // 4, _scale, 0)

        pltpu.sync_copy(e_ref, den.at[didx], add=True)
        pltpu.sync_copy(rows, acc.at[didx], add=True)
        return 0
    lax.fori_loop(0, n_chunks, _chunk, 0)

    plsc.subcore_barrier()
    pltpu.sync_copy(acc.at[pl.ds(s * rpt, rpt)],
                    outS_hbm.at[pl.ds(c * npad + s * rpt, rpt)])
    pltpu.sync_copy(den.at[pl.ds(s * rpt, rpt)],
                    outD_hbm.at[pl.ds(c * npad + s * rpt, rpt)])


def _edge_stage(h_pad, asrc_pad, adst_pad, src_pad, dst_pad):
    npad, d = h_pad.shape
    e_tot = src_pad.shape[0]
    epw = e_tot // NW
    n_chunks = epw // CHUNK
    mesh = plsc.VectorSubcoreMesh(core_axis_name="c", subcore_axis_name="s")

    outS, outD = pl.kernel(
        functools.partial(_edge_sc_body, npad, n_chunks, d),
        out_type=[jax.ShapeDtypeStruct((NC * npad, d), jnp.float32),
                  jax.ShapeDtypeStruct((NC * npad,), jnp.float32)],
        mesh=mesh,
        compiler_params=pltpu.CompilerParams(needs_layout_passes=False),
        scratch_types=[
            pltpu.VMEM((npad,), jnp.float32),
            pltpu.VMEM((npad,), jnp.float32),
            pltpu.VMEM((CHUNK,), jnp.int32),
            pltpu.VMEM((CHUNK,), jnp.int32),
            pltpu.VMEM((CHUNK,), jnp.float32),
            pltpu.VMEM((CHUNK, d), jnp.float32),
            pltpu.SemaphoreType.DMA,
            pltpu.VMEM_SHARED((npad, d), jnp.float32),
            pltpu.VMEM_SHARED((npad,), jnp.float32),
        ],
    )(h_pad, src_pad, dst_pad, asrc_pad, adst_pad)
    return outS.reshape(NC, npad, d), outD.reshape(NC, npad, 1)


def _dense1_body(x_ref, w_ref, avs_ref, avd_ref, h_ref, s_ref, d_ref):
    h = jnp.dot(x_ref[...], w_ref[...], preferred_element_type=jnp.float32)
    h_ref[...] = h
    s_ref[...] = (h @ avs_ref[...])[:, None]
    d_ref[...] = (h @ avd_ref[...])[:, None]


def _mid_body(S_ref, D_ref, b_ref, w_ref, avs_ref, avd_ref,
              h_ref, s_ref, d_ref):
    z = (S_ref[0] + S_ref[1]) / (D_ref[0] + D_ref[1] + 1e-16) + b_ref[...]
    z = jnp.maximum(z, 0.0)
    h = jnp.dot(z, w_ref[...], preferred_element_type=jnp.float32)
    h_ref[...] = h
    s_ref[...] = (h @ avs_ref[...])[:, None]
    d_ref[...] = (h @ avd_ref[...])[:, None]


def _final_body(S_ref, D_ref, b_ref, o_ref):
    o = (S_ref[0] + S_ref[1]) / (D_ref[0] + D_ref[1] + 1e-16) + b_ref[...]
    m = jnp.max(o, axis=1, keepdims=True)
    lse = jnp.log(jnp.sum(jnp.exp(o - m), axis=1, keepdims=True))
    o_ref[...] = o - m - lse


def _dense_outs(npad, d):
    return [
        jax.ShapeDtypeStruct((npad, d), jnp.float32),
        jax.ShapeDtypeStruct((npad, 1), jnp.float32),
        jax.ShapeDtypeStruct((npad, 1), jnp.float32),
    ]


def _dense_out_specs():
    return [
        pl.BlockSpec((N_BLOCK, None), lambda i: (i, 0)),
        pl.BlockSpec((N_BLOCK, 1), lambda i: (i, 0)),
        pl.BlockSpec((N_BLOCK, 1), lambda i: (i, 0)),
    ]


def _dense1(x_pad, W, att_src, att_dst):
    npad, d = x_pad.shape
    h, a_s, a_d = pl.pallas_call(
        _dense1_body,
        grid=(npad // N_BLOCK,),
        in_specs=[
            pl.BlockSpec((N_BLOCK, d), lambda i: (i, 0)),
            pl.BlockSpec((d, d), lambda i: (0, 0)),
            pl.BlockSpec((d,), lambda i: (0,)),
            pl.BlockSpec((d,), lambda i: (0,)),
        ],
        out_specs=[
            pl.BlockSpec((N_BLOCK, d), lambda i: (i, 0)),
            pl.BlockSpec((N_BLOCK, 1), lambda i: (i, 0)),
            pl.BlockSpec((N_BLOCK, 1), lambda i: (i, 0)),
        ],
        out_shape=_dense_outs(npad, d),
    )(x_pad, W, att_src, att_dst)
    return h, a_s.reshape(npad), a_d.reshape(npad)


def _mid(S, D, bias, W, att_src, att_dst):
    _, npad, d = S.shape
    h, a_s, a_d = pl.pallas_call(
        _mid_body,
        grid=(npad // N_BLOCK,),
        in_specs=[
            pl.BlockSpec((NC, N_BLOCK, d), lambda i: (0, i, 0)),
            pl.BlockSpec((NC, N_BLOCK, 1), lambda i: (0, i, 0)),
            pl.BlockSpec((d,), lambda i: (0,)),
            pl.BlockSpec((d, d), lambda i: (0, 0)),
            pl.BlockSpec((d,), lambda i: (0,)),
            pl.BlockSpec((d,), lambda i: (0,)),
        ],
        out_specs=[
            pl.BlockSpec((N_BLOCK, d), lambda i: (i, 0)),
            pl.BlockSpec((N_BLOCK, 1), lambda i: (i, 0)),
            pl.BlockSpec((N_BLOCK, 1), lambda i: (i, 0)),
        ],
        out_shape=_dense_outs(npad, d),
    )(S, D, bias, W, att_src, att_dst)
    return h, a_s.reshape(npad), a_d.reshape(npad)


def _final(S, D, bias):
    _, npad, d = S.shape
    return pl.pallas_call(
        _final_body,
        grid=(npad // N_BLOCK,),
        in_specs=[
            pl.BlockSpec((NC, N_BLOCK, d), lambda i: (0, i, 0)),
            pl.BlockSpec((NC, N_BLOCK, 1), lambda i: (0, i, 0)),
            pl.BlockSpec((d,), lambda i: (0,)),
        ],
        out_specs=pl.BlockSpec((N_BLOCK, d), lambda i: (i, 0)),
        out_shape=jax.ShapeDtypeStruct((npad, d), jnp.float32),
    )(S, D, bias)


def kernel(x, edge_index, W1, att_src1, att_dst1, b1, W2, att_src2, att_dst2, b2):
    n, d = x.shape
    e = edge_index.shape[1]
    npad = -(-(n + NS) // (NS * CHUNK)) * (NS * CHUNK)
    e_tot = e + n
    e_pad = -(-e_tot // (NW * CHUNK)) * (NW * CHUNK)

    loop = jnp.arange(n, dtype=edge_index.dtype)
    src = jnp.concatenate(
        [edge_index[0], loop,
         jnp.zeros((e_pad - e_tot,), edge_index.dtype)])
    dst = jnp.concatenate(
        [edge_index[1], loop,
         jnp.full((e_pad - e_tot,), n, edge_index.dtype)])

    x_pad = jnp.pad(x, ((0, npad - n), (0, 0)))
    h1, a1s, a1d = _dense1(x_pad, W1, att_src1, att_dst1)
    S1, D1 = _edge_stage(h1, a1s, a1d, src, dst)
    h2, a2s, a2d = _mid(S1, D1, b1, W2, att_src2, att_dst2)
    S2, D2 = _edge_stage(h2, a2s, a2d, src, dst)
    out = _final(S2, D2, b2)
    return out[:n]
